# Initial kernel scaffold; baseline (speedup 1.0000x reference)
import jax
import jax.numpy as jnp
from jax.experimental import pallas as pl


def kernel(x, edge_index, batch, params):
    raise NotImplementedError("write your pallas kernel here")


# speedup vs baseline: 3.0946x; 3.0946x over previous
import functools

import jax
import jax.numpy as jnp
from jax import lax
from jax.experimental import pallas as pl
from jax.experimental.pallas import tpu as pltpu
from jax.experimental.pallas import tpu_sc as plsc

_TW = 128
_OW = 128
_CH = 80
_NC = 2
_NS = 16


@functools.lru_cache(maxsize=None)
def _edge_kernel(n_nodes, n_edges):
    nw = _NC * _NS
    epw = n_edges // nw
    nch = epw // _CH
    assert epw % _CH == 0 and n_edges % nw == 0 and n_nodes % _CH == 0
    nrow_ch = n_nodes // _CH
    max_k = (nrow_ch + _NS - 1) // _NS
    mesh = plsc.VectorSubcoreMesh(core_axis_name="c", subcore_axis_name="s")

    @functools.partial(
        pl.kernel,
        mesh=mesh,
        out_type=jax.ShapeDtypeStruct((_NC, n_nodes, _OW), jnp.float32),
        compiler_params=pltpu.CompilerParams(needs_layout_passes=False),
        scratch_types=[
            pltpu.VMEM((_CH,), jnp.int32),
            pltpu.VMEM((_CH,), jnp.int32),
            pltpu.VMEM((_CH, _TW), jnp.float32),
            pltpu.VMEM((_CH, _TW), jnp.float32),
            pltpu.VMEM((_CH, _OW), jnp.float32),
            pltpu.VMEM((336,), jnp.float32),
            pltpu.VMEM_SHARED((n_nodes, _OW), jnp.float32),
            pltpu.SemaphoreType.DMA,
            pltpu.SemaphoreType.DMA,
        ],
    )
    def ek(td_h, ts_h, si_h, di_h, cb_h, out_h,
           sidx, didx, rd, rs, outv, cb, acc, sem1, sem2):
        cid = lax.axis_index("c")
        sid = lax.axis_index("s")
        wid = cid * _NS + sid

        i16 = lax.iota(jnp.int32, 16)
        fz = jnp.zeros((16,), jnp.float32)
        m01 = jnp.where(i16 < 2, 1.0, 0.0)
        m23 = jnp.where((i16 >= 2) & (i16 < 4), 1.0, 0.0)
        m45 = jnp.where((i16 >= 4) & (i16 < 6), 1.0, 0.0)
        oh2 = jnp.where(i16 == 2, 1.0, 0.0)

        def _zrow(i, carry):
            outv[i // 5, pl.ds((i % 5) * 16, 16)] = fz
            return carry
        lax.fori_loop(0, _CH * (_OW // 16), _zrow, 0)
        for k in range(max_k):
            c = sid + k * _NS

            @pl.when(c < nrow_ch)
            def _zero_chunk():
                off = pl.multiple_of(c * _CH, _CH)
                pltpu.sync_copy(outv, acc.at[pl.ds(off, _CH)])
        pltpu.sync_copy(cb_h, cb)
        plsc.subcore_barrier()

        bv2_s = cb[pl.ds(320, 16)][0]

        def chunk(ch, carry):
            base = wid * epw + ch * _CH
            pltpu.sync_copy(si_h.at[pl.ds(base, _CH)], sidx)
            pltpu.sync_copy(di_h.at[pl.ds(base, _CH)], didx)
            cp1 = pltpu.async_copy(td_h.at[didx], rd, sem1)
            cp2 = pltpu.async_copy(ts_h.at[sidx], rs, sem2)
            cp1.wait()
            cp2.wait()

            def edge(e, c2):
                tl = rs[e, pl.ds(96, 16)] - rd[e, pl.ds(96, 16)]
                q = tl * tl
                ds_s = jnp.sum(q * m01)
                s23 = jnp.sum(q * m23)
                s45 = jnp.sum(q * m45)
                dv_s = 0.5 * (s45 - ds_s - s23)
                for k in range(4):
                    sl = pl.ds(k * 16, 16)
                    g = (rd[e, sl] + rs[e, sl]
                         + ds_s * cb[pl.ds(64 + k * 16, 16)]
                         + dv_s * cb[pl.ds(128 + k * 16, 16)]
                         + cb[pl.ds(k * 16, 16)])
                    outv[e, sl] = g / (1.0 + jnp.exp(-g))
                u = fz
                for j in range(2):
                    sl = pl.ds(64 + j * 16, 16)
                    cc = (rd[e, sl] + rs[e, sl]
                          + ds_s * cb[pl.ds(224 + j * 16, 16)]
                          + dv_s * cb[pl.ds(256 + j * 16, 16)]
                          + cb[pl.ds(192 + j * 16, 16)])
                    u = u + (cc / (1.0 + jnp.exp(-cc))) * cb[pl.ds(288 + j * 16, 16)]
                vw = jnp.sum(u) + bv2_s
                outv[e, pl.ds(64, 16)] = vw * (tl * m01) + oh2
                return c2

            lax.fori_loop(0, _CH, edge, 0)
            pltpu.sync_copy(outv, acc.at[didx], add=True)
            return carry

        lax.fori_loop(0, nch, chunk, 0)
        plsc.subcore_barrier()
        for k in range(max_k):
            c = sid + k * _NS

            @pl.when(c < nrow_ch)
            def _copy_chunk():
                off = pl.multiple_of(c * _CH, _CH)
                pltpu.sync_copy(acc.at[pl.ds(off, _CH)],
                                out_h.at[cid, pl.ds(off, _CH)])

    return ek


def _t1_body(x_ref, xo_ref, w_ref, td_ref, ts_ref):
    xb = x_ref[...]
    n = xb.shape[0]
    p = jnp.dot(xb, w_ref[...], preferred_element_type=jnp.float32)
    pv = xo_ref[...][:, :4]
    tail = jnp.concatenate(
        [pv, pv[:, :2] + pv[:, 2:4], jnp.zeros((n, _TW - 102), jnp.float32)], axis=1)
    td_ref[...] = jnp.concatenate([p[:, :96], tail], axis=1)
    ts_ref[...] = jnp.concatenate([p[:, 96:192], tail], axis=1)


def _tables(x, x_orig, wall):
    n = x.shape[0]
    return pl.pallas_call(
        _t1_body,
        out_shape=[jax.ShapeDtypeStruct((n, _TW), jnp.float32),
                   jax.ShapeDtypeStruct((n, _TW), jnp.float32)],
    )(x, x_orig, wall)


def _t2_body(acc_ref, x_ref, we2_ref, wh1a_ref, wh1b_ref, wh2_ref, sv_ref, h_ref):
    accs = acc_ref[0] + acc_ref[1]
    xb = x_ref[...]
    sv = sv_ref[...]
    s_sum = accs[:, :64]
    mv = accs[:, 64:66]
    cnt = accs[:, 66:67]
    m_h = (jnp.dot(s_sum, we2_ref[...], preferred_element_type=jnp.float32)
           + cnt * sv[0:1, :32])
    mvn = jnp.sqrt(mv[:, :1] ** 2 + mv[:, 1:2] ** 2)
    hin = (jnp.dot(xb, wh1a_ref[...], preferred_element_type=jnp.float32)
           + jnp.dot(m_h, wh1b_ref[...], preferred_element_type=jnp.float32)
           + mvn * sv[1:2, :32] + sv[2:3, :32])
    hs = hin / (1.0 + jnp.exp(-hin))
    upd = jnp.dot(hs, wh2_ref[...], preferred_element_type=jnp.float32) + sv[3:4, :]
    r = jnp.maximum(xb + upd, 0.0)
    mu = jnp.mean(r, axis=-1, keepdims=True)
    var = jnp.mean((r - mu) ** 2, axis=-1, keepdims=True)
    h_ref[...] = sv[4:5, :] * (r - mu) / jnp.sqrt(var + 1e-5) + sv[5:6, :]


def _node_update(acc, x, we2, wh1a, wh1b, wh2, sv):
    n = x.shape[0]
    return pl.pallas_call(
        _t2_body,
        out_shape=jax.ShapeDtypeStruct((n, 128), jnp.float32),
    )(acc, x, we2, wh1a, wh1b, wh2, sv)


def _t3_body(h_ref, x_ref, bcol_ref, wp_ref, sv_ref, s_ref, p_ref, ent_ref):
    @pl.when(pl.program_id(0) == 0)
    def _init():
        p_ref[...] = jnp.zeros_like(p_ref)
        ent_ref[...] = jnp.zeros_like(ent_ref)

    hb = h_ref[...]
    sv = sv_ref[...]
    logits = jnp.dot(hb, wp_ref[...], preferred_element_type=jnp.float32) + sv[0:1, :32]
    m = jnp.max(logits, axis=-1, keepdims=True)
    ex = jnp.exp(logits - m)
    s = ex / jnp.sum(ex, axis=-1, keepdims=True)
    s_ref[...] = s
    ent_ref[...] += jnp.sum(s * jnp.log(s + 1e-10)).reshape(1, 1)
    bcol = bcol_ref[...]
    hp = jnp.concatenate([hb, x_ref[...][:, :2]], axis=1)
    zs = []
    for b in range(16):
        zs.append(s * (bcol == b).astype(jnp.float32))
    z = jnp.concatenate(zs, axis=1)
    psum = lax.dot_general(z, hp, (((0,), (0,)), ((), ())),
                           preferred_element_type=jnp.float32)
    cnts = jnp.sum(z, axis=0)[:, None]
    p_ref[...] += jnp.concatenate([psum, cnts], axis=1)


def _t4_body(n_nodes, p_ref, w1_ref, w2_ref, sv_ref, ent_in_ref, lat_ref, mu_ref, ent_ref):
    sv = sv_ref[...]
    p = p_ref[...]
    cnts = p[:, 130:131] + 1e-8
    pooled = p[:, :128] / cnts
    mu_ref[...] = (p[:, 128:130] / cnts).reshape(16, 32, 2)
    o1 = jnp.maximum(
        jnp.dot(pooled, w1_ref[...], preferred_element_type=jnp.float32) + sv[1:2, :64],
        0.0)
    lat = jnp.dot(o1, w2_ref[...], preferred_element_type=jnp.float32) + sv[2:3, :64]
    lat_ref[...] = lat.reshape(16, 32, 64)
    ent_ref[...] = -ent_in_ref[...] / n_nodes


def _pool_stage(h, x, bcol, wp, w1, w2, sv):
    n = h.shape[0]
    blk = 2000
    grid = n // blk
    s, p_acc, ent_sum = pl.pallas_call(
        _t3_body,
        grid=(grid,),
        in_specs=[pl.BlockSpec((blk, 128), lambda i: (i, 0)),
                  pl.BlockSpec((blk, 128), lambda i: (i, 0)),
                  pl.BlockSpec((blk, 1), lambda i: (i, 0)),
                  pl.BlockSpec((128, 32), lambda i: (0, 0)),
                  pl.BlockSpec((8, 128), lambda i: (0, 0))],
        out_specs=[pl.BlockSpec((blk, 32), lambda i: (i, 0)),
                   pl.BlockSpec((512, 131), lambda i: (0, 0)),
                   pl.BlockSpec((1, 1), lambda i: (0, 0))],
        out_shape=[jax.ShapeDtypeStruct((n, 32), jnp.float32),
                   jax.ShapeDtypeStruct((512, 131), jnp.float32),
                   jax.ShapeDtypeStruct((1, 1), jnp.float32)],
    )(h, x, bcol, wp, sv)
    lat, mu, ent = pl.pallas_call(
        functools.partial(_t4_body, n),
        out_shape=[jax.ShapeDtypeStruct((16, 32, 64), jnp.float32),
                   jax.ShapeDtypeStruct((16, 32, 2), jnp.float32),
                   jax.ShapeDtypeStruct((1, 1), jnp.float32)],
    )(p_acc, w1, w2, sv, ent_sum)
    return s, lat, mu, ent


def _pack_layer(p, ic):
    we1, be1 = p["e1"]["W"], p["e1"]["b"]
    wv1, bv1 = p["v1"]["W"], p["v1"]["b"]
    wall = jnp.concatenate([we1[:ic], wv1[:ic], we1[ic:2 * ic], wv1[ic:2 * ic]], axis=1)
    cb = jnp.concatenate([
        be1, we1[2 * ic], we1[2 * ic + 1],
        bv1, wv1[2 * ic], wv1[2 * ic + 1],
        p["v2"]["W"][:, 0],
        p["v2"]["b"],
        jnp.zeros((15,), jnp.float32),
    ])
    wh1 = p["h1"]["W"]
    sv = jnp.zeros((8, 128), jnp.float32)
    sv = sv.at[0, :32].set(p["e2"]["b"])
    sv = sv.at[1, :32].set(wh1[ic + 32])
    sv = sv.at[2, :32].set(p["h1"]["b"])
    sv = sv.at[3, :].set(p["h2"]["b"])
    return (wall, cb, p["e2"]["W"], wh1[:ic], wh1[ic:ic + 32], p["h2"]["W"], sv)


def _gnn_layer(ek, x, x_orig, src, dst, packed, ln_g, ln_b):
    wall, cb, we2, wh1a, wh1b, wh2, sv = packed
    sv = sv.at[4, :].set(ln_g).at[5, :].set(ln_b)
    tdst, tsrc = _tables(x, x_orig, wall)
    acc = ek(tdst, tsrc, src, dst, cb)
    return _node_update(acc, x, we2, wh1a, wh1b, wh2, sv)


def kernel(x, edge_index, batch, params):
    n = x.shape[0]
    e = edge_index.shape[1]
    src = edge_index[0]
    dst = edge_index[1]
    ek = _edge_kernel(n, e)

    h = _gnn_layer(ek, x, x, src, dst, _pack_layer(params["gnn1"], 128),
                   params["ln1_g"], params["ln1_b"])
    h = _gnn_layer(ek, h, x, src, dst, _pack_layer(params["gnn2"], 128),
                   params["ln2_g"], params["ln2_b"])

    sv3 = jnp.zeros((8, 128), jnp.float32)
    sv3 = sv3.at[0, :32].set(params["pool"]["b"])
    sv3 = sv3.at[1, :64].set(params["out1"]["b"])
    sv3 = sv3.at[2, :64].set(params["out2"]["b"])
    s, lat, mu, ent = _pool_stage(h, x, batch.reshape(n, 1),
                                  params["pool"]["W"], params["out1"]["W"],
                                  params["out2"]["W"], sv3)
    return lat, s, ent.reshape(()), mu


# speedup vs baseline: 9.0751x; 2.9326x over previous
import functools

import jax
import jax.numpy as jnp
from jax import lax
from jax.experimental import pallas as pl
from jax.experimental.pallas import tpu as pltpu
from jax.experimental.pallas import tpu_sc as plsc

_TW = 128
_OW = 128
_CH = 80
_NC = 2
_NS = 16


@functools.lru_cache(maxsize=None)
def _edge_kernel(n_nodes, n_edges):
    nw = _NC * _NS
    epw = n_edges // nw
    nch = epw // _CH
    assert epw % _CH == 0 and n_edges % nw == 0 and n_nodes % _CH == 0
    nrow_ch = n_nodes // _CH
    max_k = (nrow_ch + _NS - 1) // _NS
    mesh = plsc.VectorSubcoreMesh(core_axis_name="c", subcore_axis_name="s")

    @functools.partial(
        pl.kernel,
        mesh=mesh,
        out_type=jax.ShapeDtypeStruct((_NC, n_nodes, _OW), jnp.float32),
        compiler_params=pltpu.CompilerParams(needs_layout_passes=False),
        scratch_types=[
            pltpu.VMEM((_CH,), jnp.int32),
            pltpu.VMEM((_CH,), jnp.int32),
            pltpu.VMEM((_CH, _TW), jnp.float32),
            pltpu.VMEM((_CH, _TW), jnp.float32),
            pltpu.VMEM((_CH, _OW), jnp.float32),
            pltpu.VMEM((336,), jnp.float32),
            pltpu.VMEM_SHARED((n_nodes, _OW), jnp.float32),
            pltpu.SemaphoreType.DMA,
            pltpu.SemaphoreType.DMA,
        ],
    )
    def ek(td_h, ts_h, si_h, di_h, cb_h, out_h,
           sidx, didx, rd, rs, outv, cb, acc, sem1, sem2):
        cid = lax.axis_index("c")
        sid = lax.axis_index("s")
        wid = cid * _NS + sid

        i16 = lax.iota(jnp.int32, 16)
        fz = jnp.zeros((16,), jnp.float32)
        m01 = jnp.where(i16 < 2, 1.0, 0.0)
        m23 = jnp.where((i16 >= 2) & (i16 < 4), 1.0, 0.0)
        m45 = jnp.where((i16 >= 4) & (i16 < 6), 1.0, 0.0)
        oh2 = jnp.where(i16 == 2, 1.0, 0.0)

        def _zrow(i, carry):
            outv[i // 5, pl.ds((i % 5) * 16, 16)] = fz
            return carry
        lax.fori_loop(0, _CH * (_OW // 16), _zrow, 0)
        for k in range(max_k):
            c = sid + k * _NS

            @pl.when(c < nrow_ch)
            def _zero_chunk():
                off = pl.multiple_of(c * _CH, _CH)
                pltpu.sync_copy(outv, acc.at[pl.ds(off, _CH)])
        pltpu.sync_copy(cb_h, cb)
        plsc.subcore_barrier()

        bv2_s = cb[pl.ds(320, 16)][0]

        def chunk(ch, carry):
            base = wid * epw + ch * _CH
            pltpu.sync_copy(si_h.at[pl.ds(base, _CH)], sidx)
            pltpu.sync_copy(di_h.at[pl.ds(base, _CH)], didx)
            cp1 = pltpu.async_copy(td_h.at[didx], rd, sem1)
            cp2 = pltpu.async_copy(ts_h.at[sidx], rs, sem2)
            cp1.wait()
            cp2.wait()

            @plsc.parallel_loop(0, _CH, 1, unroll=4)
            def edge(e):
                tl = rs[e, pl.ds(96, 16)] - rd[e, pl.ds(96, 16)]
                cs = jnp.cumsum(tl * tl)
                ds_s = cs[1]
                dv_s = 0.5 * (cs[5] - 2.0 * cs[3])
                for k in range(4):
                    sl = pl.ds(k * 16, 16)
                    g = (rd[e, sl] + rs[e, sl]
                         + ds_s * cb[pl.ds(64 + k * 16, 16)]
                         + dv_s * cb[pl.ds(128 + k * 16, 16)]
                         + cb[pl.ds(k * 16, 16)])
                    outv[e, sl] = g / (1.0 + jnp.exp(-g))
                u = fz
                for j in range(2):
                    sl = pl.ds(64 + j * 16, 16)
                    cc = (rd[e, sl] + rs[e, sl]
                          + ds_s * cb[pl.ds(224 + j * 16, 16)]
                          + dv_s * cb[pl.ds(256 + j * 16, 16)]
                          + cb[pl.ds(192 + j * 16, 16)])
                    u = u + (cc / (1.0 + jnp.exp(-cc))) * cb[pl.ds(288 + j * 16, 16)]
                vw = jnp.sum(u) + bv2_s
                outv[e, pl.ds(64, 16)] = vw * (tl * m01) + oh2

            pltpu.sync_copy(outv, acc.at[didx], add=True)
            return carry

        lax.fori_loop(0, nch, chunk, 0)
        plsc.subcore_barrier()
        for k in range(max_k):
            c = sid + k * _NS

            @pl.when(c < nrow_ch)
            def _copy_chunk():
                off = pl.multiple_of(c * _CH, _CH)
                pltpu.sync_copy(acc.at[pl.ds(off, _CH)],
                                out_h.at[cid, pl.ds(off, _CH)])

    return ek


def _t1_body(x_ref, xo_ref, w_ref, td_ref, ts_ref):
    xb = x_ref[...]
    n = xb.shape[0]
    p = jnp.dot(xb, w_ref[...], preferred_element_type=jnp.float32)
    pv = xo_ref[...][:, :4]
    tail = jnp.concatenate(
        [pv, pv[:, :2] + pv[:, 2:4], jnp.zeros((n, _TW - 102), jnp.float32)], axis=1)
    td_ref[...] = jnp.concatenate([p[:, :96], tail], axis=1)
    ts_ref[...] = jnp.concatenate([p[:, 96:192], tail], axis=1)


def _tables(x, x_orig, wall):
    n = x.shape[0]
    return pl.pallas_call(
        _t1_body,
        out_shape=[jax.ShapeDtypeStruct((n, _TW), jnp.float32),
                   jax.ShapeDtypeStruct((n, _TW), jnp.float32)],
    )(x, x_orig, wall)


def _t2_body(acc_ref, x_ref, we2_ref, wh1a_ref, wh1b_ref, wh2_ref, sv_ref, h_ref):
    accs = acc_ref[0] + acc_ref[1]
    xb = x_ref[...]
    sv = sv_ref[...]
    s_sum = accs[:, :64]
    mv = accs[:, 64:66]
    cnt = accs[:, 66:67]
    m_h = (jnp.dot(s_sum, we2_ref[...], preferred_element_type=jnp.float32)
           + cnt * sv[0:1, :32])
    mvn = jnp.sqrt(mv[:, :1] ** 2 + mv[:, 1:2] ** 2)
    hin = (jnp.dot(xb, wh1a_ref[...], preferred_element_type=jnp.float32)
           + jnp.dot(m_h, wh1b_ref[...], preferred_element_type=jnp.float32)
           + mvn * sv[1:2, :32] + sv[2:3, :32])
    hs = hin / (1.0 + jnp.exp(-hin))
    upd = jnp.dot(hs, wh2_ref[...], preferred_element_type=jnp.float32) + sv[3:4, :]
    r = jnp.maximum(xb + upd, 0.0)
    mu = jnp.mean(r, axis=-1, keepdims=True)
    var = jnp.mean((r - mu) ** 2, axis=-1, keepdims=True)
    h_ref[...] = sv[4:5, :] * (r - mu) / jnp.sqrt(var + 1e-5) + sv[5:6, :]


def _node_update(acc, x, we2, wh1a, wh1b, wh2, sv):
    n = x.shape[0]
    return pl.pallas_call(
        _t2_body,
        out_shape=jax.ShapeDtypeStruct((n, 128), jnp.float32),
    )(acc, x, we2, wh1a, wh1b, wh2, sv)


def _t3_body(h_ref, x_ref, bcol_ref, wp_ref, sv_ref, s_ref, p_ref, ent_ref):
    @pl.when(pl.program_id(0) == 0)
    def _init():
        p_ref[...] = jnp.zeros_like(p_ref)
        ent_ref[...] = jnp.zeros_like(ent_ref)

    hb = h_ref[...]
    sv = sv_ref[...]
    logits = jnp.dot(hb, wp_ref[...], preferred_element_type=jnp.float32) + sv[0:1, :32]
    m = jnp.max(logits, axis=-1, keepdims=True)
    ex = jnp.exp(logits - m)
    s = ex / jnp.sum(ex, axis=-1, keepdims=True)
    s_ref[...] = s
    ent_ref[...] += jnp.sum(s * jnp.log(s + 1e-10)).reshape(1, 1)
    bcol = bcol_ref[...]
    hp = jnp.concatenate([hb, x_ref[...][:, :2]], axis=1)
    zs = []
    for b in range(16):
        zs.append(s * (bcol == b).astype(jnp.float32))
    z = jnp.concatenate(zs, axis=1)
    psum = lax.dot_general(z, hp, (((0,), (0,)), ((), ())),
                           preferred_element_type=jnp.float32)
    cnts = jnp.sum(z, axis=0)[:, None]
    p_ref[...] += jnp.concatenate([psum, cnts], axis=1)


def _t4_body(n_nodes, p_ref, w1_ref, w2_ref, sv_ref, ent_in_ref, lat_ref, mu_ref, ent_ref):
    sv = sv_ref[...]
    p = p_ref[...]
    cnts = p[:, 130:131] + 1e-8
    pooled = p[:, :128] / cnts
    mu_ref[...] = (p[:, 128:130] / cnts).reshape(16, 32, 2)
    o1 = jnp.maximum(
        jnp.dot(pooled, w1_ref[...], preferred_element_type=jnp.float32) + sv[1:2, :64],
        0.0)
    lat = jnp.dot(o1, w2_ref[...], preferred_element_type=jnp.float32) + sv[2:3, :64]
    lat_ref[...] = lat.reshape(16, 32, 64)
    ent_ref[...] = -ent_in_ref[...] / n_nodes


def _pool_stage(h, x, bcol, wp, w1, w2, sv):
    n = h.shape[0]
    blk = 2000
    grid = n // blk
    s, p_acc, ent_sum = pl.pallas_call(
        _t3_body,
        grid=(grid,),
        in_specs=[pl.BlockSpec((blk, 128), lambda i: (i, 0)),
                  pl.BlockSpec((blk, 128), lambda i: (i, 0)),
                  pl.BlockSpec((blk, 1), lambda i: (i, 0)),
                  pl.BlockSpec((128, 32), lambda i: (0, 0)),
                  pl.BlockSpec((8, 128), lambda i: (0, 0))],
        out_specs=[pl.BlockSpec((blk, 32), lambda i: (i, 0)),
                   pl.BlockSpec((512, 131), lambda i: (0, 0)),
                   pl.BlockSpec((1, 1), lambda i: (0, 0))],
        out_shape=[jax.ShapeDtypeStruct((n, 32), jnp.float32),
                   jax.ShapeDtypeStruct((512, 131), jnp.float32),
                   jax.ShapeDtypeStruct((1, 1), jnp.float32)],
    )(h, x, bcol, wp, sv)
    lat, mu, ent = pl.pallas_call(
        functools.partial(_t4_body, n),
        out_shape=[jax.ShapeDtypeStruct((16, 32, 64), jnp.float32),
                   jax.ShapeDtypeStruct((16, 32, 2), jnp.float32),
                   jax.ShapeDtypeStruct((1, 1), jnp.float32)],
    )(p_acc, w1, w2, sv, ent_sum)
    return s, lat, mu, ent


def _pack_layer(p, ic):
    we1, be1 = p["e1"]["W"], p["e1"]["b"]
    wv1, bv1 = p["v1"]["W"], p["v1"]["b"]
    wall = jnp.concatenate([we1[:ic], wv1[:ic], we1[ic:2 * ic], wv1[ic:2 * ic]], axis=1)
    cb = jnp.concatenate([
        be1, we1[2 * ic], we1[2 * ic + 1],
        bv1, wv1[2 * ic], wv1[2 * ic + 1],
        p["v2"]["W"][:, 0],
        p["v2"]["b"],
        jnp.zeros((15,), jnp.float32),
    ])
    wh1 = p["h1"]["W"]
    sv = jnp.zeros((8, 128), jnp.float32)
    sv = sv.at[0, :32].set(p["e2"]["b"])
    sv = sv.at[1, :32].set(wh1[ic + 32])
    sv = sv.at[2, :32].set(p["h1"]["b"])
    sv = sv.at[3, :].set(p["h2"]["b"])
    return (wall, cb, p["e2"]["W"], wh1[:ic], wh1[ic:ic + 32], p["h2"]["W"], sv)


def _gnn_layer(ek, x, x_orig, src, dst, packed, ln_g, ln_b):
    wall, cb, we2, wh1a, wh1b, wh2, sv = packed
    sv = sv.at[4, :].set(ln_g).at[5, :].set(ln_b)
    tdst, tsrc = _tables(x, x_orig, wall)
    acc = ek(tdst, tsrc, src, dst, cb)
    return _node_update(acc, x, we2, wh1a, wh1b, wh2, sv)


def kernel(x, edge_index, batch, params):
    n = x.shape[0]
    e = edge_index.shape[1]
    src = edge_index[0]
    dst = edge_index[1]
    ek = _edge_kernel(n, e)

    h = _gnn_layer(ek, x, x, src, dst, _pack_layer(params["gnn1"], 128),
                   params["ln1_g"], params["ln1_b"])
    h = _gnn_layer(ek, h, x, src, dst, _pack_layer(params["gnn2"], 128),
                   params["ln2_g"], params["ln2_b"])

    sv3 = jnp.zeros((8, 128), jnp.float32)
    sv3 = sv3.at[0, :32].set(params["pool"]["b"])
    sv3 = sv3.at[1, :64].set(params["out1"]["b"])
    sv3 = sv3.at[2, :64].set(params["out2"]["b"])
    s, lat, mu, ent = _pool_stage(h, x, batch.reshape(n, 1),
                                  params["pool"]["W"], params["out1"]["W"],
                                  params["out2"]["W"], sv3)
    return lat, s, ent.reshape(()), mu
